# Initial kernel scaffold; baseline (speedup 1.0000x reference)
import functools

import jax
import jax.numpy as jnp
from jax import lax
from jax.experimental import pallas as pl
from jax.experimental.pallas import tpu as pltpu
from jax.experimental.pallas import tpu_sc as plsc

N_NODES = 10000
N_EDGES = 320000
D = 128

NC = 2
NS = 16
NW = NC * NS

NPAD = 10240
RPT = NPAD // NS
C = 80
EPW = N_EDGES // NW
DEGW = 16


def _sc_aggregate(x, src, dst):
    mesh = plsc.VectorSubcoreMesh(core_axis_name="c", subcore_axis_name="s")

    @functools.partial(
        pl.kernel,
        out_type=(
            jax.ShapeDtypeStruct((NC * NPAD, D), jnp.float32),
            jax.ShapeDtypeStruct((NC * NPAD, DEGW), jnp.float32),
        ),
        mesh=mesh,
        scratch_types=[
            pltpu.VMEM((C, D), jnp.float32),
            pltpu.VMEM((C, DEGW), jnp.float32),
            pltpu.VMEM((C, DEGW), jnp.float32),
            pltpu.VMEM((C,), jnp.int32),
            pltpu.VMEM((C,), jnp.int32),
            pltpu.VMEM((C, D), jnp.float32),
            pltpu.SemaphoreType.DMA,
            pltpu.VMEM_SHARED((NPAD, D), jnp.float32),
            pltpu.VMEM_SHARED((NPAD, DEGW), jnp.float32),
        ],
    )
    def agg_kernel(x_hbm, src_hbm, dst_hbm, sum_out, deg_out,
                   zbuf, zdeg, ones, sidx, didx, rows, sem, acc, dacc):
        c = lax.axis_index("c")
        s = lax.axis_index("s")
        w = c * NS + s

        zero16 = jnp.zeros((16,), jnp.float32)
        onehot = jnp.where(lax.iota(jnp.int32, 16) == 0, 1.0, 0.0)

        @pl.loop(0, C)
        def _fill(i):
            for j in range(D // 16):
                zbuf[i, pl.ds(j * 16, 16)] = zero16
            zdeg[i, pl.ds(0, DEGW)] = zero16
            ones[i, pl.ds(0, DEGW)] = onehot

        @pl.loop(0, RPT // C)
        def _zero(k):
            base = s * RPT + k * C
            pltpu.sync_copy(zbuf, acc.at[pl.ds(base, C)])
            pltpu.sync_copy(zdeg, dacc.at[pl.ds(base, C)])

        plsc.subcore_barrier()

        @pl.loop(0, EPW // C)
        def _edges(it):
            base = w * EPW + it * C
            pltpu.sync_copy(src_hbm.at[pl.ds(base, C)], sidx)
            pltpu.sync_copy(dst_hbm.at[pl.ds(base, C)], didx)
            pltpu.async_copy(x_hbm.at[sidx], rows, sem).wait()
            pltpu.sync_copy(rows, acc.at[didx], add=True)
            pltpu.sync_copy(ones, dacc.at[didx], add=True)

        plsc.subcore_barrier()

        out_base = c * NPAD + s * RPT
        pltpu.sync_copy(acc.at[pl.ds(s * RPT, RPT)],
                        sum_out.at[pl.ds(out_base, RPT)])
        pltpu.sync_copy(dacc.at[pl.ds(s * RPT, RPT)],
                        deg_out.at[pl.ds(out_base, RPT)])

    return agg_kernel(x, src, dst)


BR = 256


def _tc_combine_body(x_ref, s0_ref, s1_ref, d0_ref, d1_ref, w_ref, b_ref,
                     o_ref):
    total = s0_ref[...] + s1_ref[...] + x_ref[...]
    deg = d0_ref[:, 0:1] + d1_ref[:, 0:1] + 1.0
    agg = total / deg
    h = jnp.dot(agg, w_ref[...], preferred_element_type=jnp.float32)
    h = jnp.maximum(h + b_ref[...], 0.0)
    n = jnp.sqrt(jnp.sum(h * h, axis=1, keepdims=True))
    o_ref[...] = h / jnp.maximum(n, 1e-12)


def _tc_combine(xpad, sums, degs, W, b2):
    grid = NPAD // BR
    return pl.pallas_call(
        _tc_combine_body,
        grid=(grid,),
        in_specs=[
            pl.BlockSpec((BR, D), lambda i: (i, 0)),
            pl.BlockSpec((BR, D), lambda i: (i, 0)),
            pl.BlockSpec((BR, D), lambda i, _g=grid: (i + _g, 0)),
            pl.BlockSpec((BR, DEGW), lambda i: (i, 0)),
            pl.BlockSpec((BR, DEGW), lambda i, _g=grid: (i + _g, 0)),
            pl.BlockSpec((D, D), lambda i: (0, 0)),
            pl.BlockSpec((1, D), lambda i: (0, 0)),
        ],
        out_specs=pl.BlockSpec((BR, D), lambda i: (i, 0)),
        out_shape=jax.ShapeDtypeStruct((NPAD, D), jnp.float32),
    )(xpad, sums, sums, degs, degs, W, b2)


def kernel(x, edge_index, W, b):
    src = edge_index[0]
    dst = edge_index[1]
    sums, degs = _sc_aggregate(x, src, dst)
    xpad = jnp.pad(x, ((0, NPAD - N_NODES), (0, 0)))
    h = _tc_combine(xpad, sums, degs, W, b.reshape(1, D))
    return h[:N_NODES]


# speedup vs baseline: 5.0591x; 5.0591x over previous
import functools

import jax
import jax.numpy as jnp
from jax import lax
from jax.experimental import pallas as pl
from jax.experimental.pallas import tpu as pltpu
from jax.experimental.pallas import tpu_sc as plsc

N_NODES = 10000
N_EDGES = 320000
D = 128

NC = 2
NS = 16
NW = NC * NS

NPAD = 10240
RPT = NPAD // NS
C = 80
EPW = N_EDGES // NW
DEGW = 16


def _mesh():
    return plsc.VectorSubcoreMesh(core_axis_name="c", subcore_axis_name="s",
                                  num_cores=NC, num_subcores=NS)


def _sc_sums(x, src, dst):
    @functools.partial(
        pl.kernel,
        out_type=jax.ShapeDtypeStruct((NC * NPAD, D), jnp.float32),
        mesh=_mesh(),
        scratch_types=[
            pltpu.VMEM((C, D), jnp.float32),
            pltpu.VMEM((C,), jnp.int32),
            pltpu.VMEM((C,), jnp.int32),
            pltpu.VMEM((C, D), jnp.float32),
            pltpu.SemaphoreType.DMA,
            pltpu.VMEM_SHARED((NPAD, D), jnp.float32),
        ],
    )
    def sums_kernel(x_hbm, src_hbm, dst_hbm, sum_out,
                    zbuf, sidx, didx, rows, sem, acc):
        c = lax.axis_index("c")
        s = lax.axis_index("s")
        w = c * NS + s

        zero16 = jnp.zeros((16,), jnp.float32)

        @pl.loop(0, C)
        def _fill(i):
            for j in range(D // 16):
                zbuf[i, pl.ds(j * 16, 16)] = zero16

        @pl.loop(0, RPT // C)
        def _zero(k):
            pltpu.sync_copy(zbuf, acc.at[pl.ds(s * RPT + k * C, C)])

        plsc.subcore_barrier()

        @pl.loop(0, EPW // C)
        def _edges(it):
            base = w * EPW + it * C
            pltpu.sync_copy(src_hbm.at[pl.ds(base, C)], sidx)
            pltpu.sync_copy(dst_hbm.at[pl.ds(base, C)], didx)
            pltpu.async_copy(x_hbm.at[sidx], rows, sem).wait()
            pltpu.sync_copy(rows, acc.at[didx], add=True)

        plsc.subcore_barrier()

        pltpu.sync_copy(acc.at[pl.ds(s * RPT, RPT)],
                        sum_out.at[pl.ds(c * NPAD + s * RPT, RPT)])

    return sums_kernel(x, src, dst)


def _sc_degree(dst):
    @functools.partial(
        pl.kernel,
        out_type=jax.ShapeDtypeStruct((NC * NPAD, DEGW), jnp.float32),
        mesh=_mesh(),
        scratch_types=[
            pltpu.VMEM((C, DEGW), jnp.float32),
            pltpu.VMEM((C, DEGW), jnp.float32),
            pltpu.VMEM((C,), jnp.int32),
            pltpu.VMEM_SHARED((NPAD, DEGW), jnp.float32),
        ],
    )
    def deg_kernel(dst_hbm, deg_out, zdeg, ones, didx, dacc):
        c = lax.axis_index("c")
        s = lax.axis_index("s")
        w = c * NS + s

        zero16 = jnp.zeros((16,), jnp.float32)
        onehot = jnp.where(lax.iota(jnp.int32, 16) == 0,
                           jnp.float32(1), jnp.float32(0))

        @pl.loop(0, C)
        def _fill(i):
            zdeg[i, pl.ds(0, DEGW)] = zero16
            ones[i, pl.ds(0, DEGW)] = onehot

        @pl.loop(0, RPT // C)
        def _zero(k):
            pltpu.sync_copy(zdeg, dacc.at[pl.ds(s * RPT + k * C, C)])

        plsc.subcore_barrier()

        @pl.loop(0, EPW // C)
        def _edges(it):
            pltpu.sync_copy(dst_hbm.at[pl.ds(w * EPW + it * C, C)], didx)
            pltpu.sync_copy(ones, dacc.at[didx], add=True)

        plsc.subcore_barrier()

        pltpu.sync_copy(dacc.at[pl.ds(s * RPT, RPT)],
                        deg_out.at[pl.ds(c * NPAD + s * RPT, RPT)])

    return deg_kernel(dst)


BR = 256


def _tc_combine_body(x_ref, s0_ref, s1_ref, d0_ref, d1_ref, w_ref, b_ref,
                     o_ref):
    total = s0_ref[...] + s1_ref[...] + x_ref[...]
    deg = d0_ref[:, 0:1] + d1_ref[:, 0:1] + 1.0
    agg = total / deg
    h = jnp.dot(agg, w_ref[...], preferred_element_type=jnp.float32)
    h = jnp.maximum(h + b_ref[...], 0.0)
    n = jnp.sqrt(jnp.sum(h * h, axis=1, keepdims=True))
    o_ref[...] = h / jnp.maximum(n, 1e-12)


def _tc_combine(xpad, sums, degs, W, b2):
    grid = NPAD // BR
    return pl.pallas_call(
        _tc_combine_body,
        grid=(grid,),
        in_specs=[
            pl.BlockSpec((BR, D), lambda i: (i, 0)),
            pl.BlockSpec((BR, D), lambda i: (i, 0)),
            pl.BlockSpec((BR, D), lambda i, _g=grid: (i + _g, 0)),
            pl.BlockSpec((BR, DEGW), lambda i: (i, 0)),
            pl.BlockSpec((BR, DEGW), lambda i, _g=grid: (i + _g, 0)),
            pl.BlockSpec((D, D), lambda i: (0, 0)),
            pl.BlockSpec((1, D), lambda i: (0, 0)),
        ],
        out_specs=pl.BlockSpec((BR, D), lambda i: (i, 0)),
        out_shape=jax.ShapeDtypeStruct((NPAD, D), jnp.float32),
    )(xpad, sums, sums, degs, degs, W, b2)


def kernel(x, edge_index, W, b):
    src = edge_index[0]
    dst = edge_index[1]
    sums = _sc_sums(x, src, dst)
    degs = _sc_degree(dst)
    xpad = jnp.pad(x, ((0, NPAD - N_NODES), (0, 0)))
    h = _tc_combine(xpad, sums, degs, W, b.reshape(1, D))
    return h[:N_NODES]
